# baseline (speedup 1.0000x reference)
import functools

import jax
import jax.numpy as jnp
from jax.experimental import pallas as pl
from jax.experimental.pallas import tpu as pltpu

_BR = 1024


def _select_kernel(y_ref, x_ref, o_ref):
    x = x_ref[...]
    yb = y_ref[...].reshape(_BR, 1)
    ids = jax.lax.broadcasted_iota(jnp.int32, x.shape, 1)
    o_ref[...] = jnp.sum(jnp.where(ids == yb, x, 0.0), axis=1)


def kernel(logits, y):
    B, C = logits.shape
    y32 = y.astype(jnp.int32)
    grid = (B // _BR,)
    return pl.pallas_call(
        _select_kernel,
        grid=grid,
        in_specs=[
            pl.BlockSpec((_BR,), lambda i: (i,)),
            pl.BlockSpec((_BR, C), lambda i: (i, 0)),
        ],
        out_specs=pl.BlockSpec((_BR,), lambda i: (i,)),
        out_shape=jax.ShapeDtypeStruct((B,), jnp.float32),
    )(y32, logits)
